# baseline (speedup 1.0000x reference)
import jax
import jax.numpy as jnp
from jax import lax
from jax.experimental import pallas as pl
from jax.experimental.pallas import tpu as pltpu
from jax.experimental.pallas import tpu_sc as plsc

BATCH = 4096
SEQ_LEN = 200
HIDDEN = 64
LANES = 16

NUM_CORES = 2
NUM_SUBCORES = 16
NUM_WORKERS = NUM_CORES * NUM_SUBCORES

LT = SEQ_LEN // 8

_TAKE_DNUMS = lax.GatherDimensionNumbers(
    offset_dims=(), collapsed_slice_dims=(0,), start_index_map=(0,)
)


def _take16(vec, idx):
    return lax.gather(
        vec,
        idx[:, None],
        _TAKE_DNUMS,
        slice_sizes=(1,),
        mode=lax.GatherScatterMode.PROMISE_IN_BOUNDS,
    )


NSLAB = 999936 // 256


def _pair_body(wtp_hbm, t2_hbm, in0, in1, ot0, ot1, semi0, semi1, semt0, semt1):
    wid = lax.axis_index("s") * NUM_CORES + lax.axis_index("c")
    inb = (in0, in1)
    otb = (ot0, ot1)
    semi = (semi0, semi1)
    semt = (semt0, semt1)

    def slab_of(i):
        return wid + i * NUM_WORKERS

    def issue_in(i, b):
        s = slab_of(i)

        @pl.when(s < NSLAB)
        def _():
            pltpu.async_copy(wtp_hbm.at[pl.ds(s * 256, 256)], inb[b], semi[b])

    def wait_in(i, b):
        s = slab_of(i)

        @pl.when(s < NSLAB)
        def _():
            pltpu.make_async_copy(wtp_hbm.at[pl.ds(0, 256)], inb[b], semi[b]).wait()

    def wait_out(i, b):
        s = slab_of(i)

        @pl.when(s < NSLAB)
        def _():
            pltpu.make_async_copy(otb[b], t2_hbm.at[pl.ds(0, 128)], semt[b]).wait()

    issue_in(0, 0)
    issue_in(1, 1)

    def super_body(sp, carry):
        for b in range(2):
            i = 2 * sp + b
            s = slab_of(i)
            wait_in(i, b)

            @pl.when(i >= 2)
            def _():
                wait_out(i - 2, b)

            @pl.when(s < NSLAB)
            def _():
                @plsc.parallel_loop(0, 128, unroll=4)
                def _p(p):
                    for q in range(4):
                        otb[b][p, pl.ds(16 * q, 16)] = inb[b][2 * p, pl.ds(16 * q, 16)]
                        otb[b][p, pl.ds(64 + 16 * q, 16)] = inb[b][2 * p + 1, pl.ds(16 * q, 16)]

                pltpu.async_copy(otb[b], t2_hbm.at[pl.ds(s * 128, 128)], semt[b])

            issue_in(i + 2, b)
        return carry

    lax.fori_loop(0, 62, super_body, 0)
    wait_out(122, 0)

    @pl.when(wid == 0)
    def _():
        pltpu.async_copy(wtp_hbm.at[pl.ds(999936, 64)], in0.at[pl.ds(0, 64)], semi0)
        pltpu.make_async_copy(
            wtp_hbm.at[pl.ds(0, 64)], in0.at[pl.ds(0, 64)], semi0
        ).wait()

        @plsc.parallel_loop(0, 32, unroll=4)
        def _p(p):
            for q in range(4):
                ot0[p, pl.ds(16 * q, 16)] = in0[2 * p, pl.ds(16 * q, 16)]
                ot0[p, pl.ds(64 + 16 * q, 16)] = in0[2 * p + 1, pl.ds(16 * q, 16)]

        pltpu.async_copy(ot0.at[pl.ds(0, 32)], t2_hbm.at[pl.ds(499968, 32)], semt0)
        pltpu.make_async_copy(
            ot0.at[pl.ds(0, 32)], t2_hbm.at[pl.ds(0, 32)], semt0
        ).wait()


def _body(xt_hbm, wt2_hbm, px_hbm, out_hbm, idx_v, idx2_v, posall_v,
          rows0, rows1, rows2, rows3, st0, st1,
          semg0, semg1, semg2, semg3, semo0, semo1):
    wid = lax.axis_index("s") * NUM_CORES + lax.axis_index("c")
    b0 = wid * 128

    rows = (rows0, rows1, rows2, rows3)
    stage = (st0, st1)
    semg = (semg0, semg1, semg2, semg3)
    semo = (semo0, semo1)

    iota = lax.iota(jnp.int32, LANES)
    rowvec = [iota + (16 * k) for k in range(8)]

    def issue_gather(j, b):
        pltpu.async_copy(wt2_hbm.at[idx2_v.at[j]], rows[b], semg[b])

    def wait_gather(b):
        pltpu.make_async_copy(wt2_hbm.at[pl.ds(0, 128)], rows[b], semg[b]).wait()

    def wait_out(b):
        pltpu.make_async_copy(stage[b], out_hbm.at[0, :, pl.ds(b0, 128)], semo[b]).wait()

    pltpu.sync_copy(px_hbm, posall_v)

    def lt_body(lt, carry):
        l0 = lt * 8
        pltpu.sync_copy(xt_hbm.at[pl.ds(l0, 8), pl.ds(b0, 128)], idx_v)
        for j in range(8):
            for k in range(8):
                v16 = idx_v[j, pl.ds(16 * k, 16)]
                idx2_v[j, pl.ds(16 * k, 16)] = lax.shift_right_logical(v16, 1)
        for j3 in range(3):
            issue_gather(j3, j3)
        for j in range(8):
            if j < 5:
                issue_gather(j + 3, (j + 3) % 4)
            b = j % 4
            s = j % 2
            wait_gather(b)
            if j >= 2:
                wait_out(s)
            else:
                @pl.when(lt > 0)
                def _():
                    wait_out(s)
            pv = [
                lax.shift_left(idx_v[j, pl.ds(16 * k, 16)] & 1, 6)
                for k in range(8)
            ]

            @plsc.parallel_loop(0, HIDDEN, unroll=4)
            def _diag_loop(i):
                d = i & 15
                h0 = lax.shift_right_logical(i, 4) * 16
                rot = (iota + d) & 15
                hvec = rot + h0
                povec = posall_v[l0 + j, pl.ds(h0, 16)]
                ps = _take16(povec, rot)
                for k in range(8):
                    col = pv[k] + hvec
                    vec = plsc.load_gather(rows[b], [rowvec[k], col])
                    plsc.store_scatter(stage[s], [hvec, rowvec[k]], vec + ps)
            pltpu.async_copy(stage[s], out_hbm.at[l0 + j, :, pl.ds(b0, 128)], semo[s])
        return carry

    lax.fori_loop(0, LT, lt_body, 0)
    wait_out(0)
    wait_out(1)


@jax.jit
def _run(xt, wtp, pt):
    mesh = plsc.VectorSubcoreMesh(core_axis_name="c", subcore_axis_name="s")
    wt2 = pl.kernel(
        _pair_body,
        out_type=jax.ShapeDtypeStruct((500000, 128), jnp.float32),
        mesh=mesh,
        compiler_params=pltpu.CompilerParams(
            use_tc_tiling_on_sc=True, needs_layout_passes=False
        ),
        scratch_types=[
            pltpu.VMEM((256, 64), jnp.float32),
            pltpu.VMEM((256, 64), jnp.float32),
            pltpu.VMEM((128, 128), jnp.float32),
            pltpu.VMEM((128, 128), jnp.float32),
            pltpu.SemaphoreType.DMA,
            pltpu.SemaphoreType.DMA,
            pltpu.SemaphoreType.DMA,
            pltpu.SemaphoreType.DMA,
        ],
    )(wtp)
    return pl.kernel(
        _body,
        out_type=jax.ShapeDtypeStruct((SEQ_LEN, HIDDEN, BATCH), jnp.float32),
        mesh=mesh,
        compiler_params=pltpu.CompilerParams(
            use_tc_tiling_on_sc=True, needs_layout_passes=False
        ),
        scratch_types=[
            pltpu.VMEM((8, 128), jnp.int32),
            pltpu.VMEM((8, 128), jnp.int32),
            pltpu.VMEM((SEQ_LEN, 128), jnp.float32),
            pltpu.VMEM((128, 128), jnp.float32),
            pltpu.VMEM((128, 128), jnp.float32),
            pltpu.VMEM((128, 128), jnp.float32),
            pltpu.VMEM((128, 128), jnp.float32),
            pltpu.VMEM((HIDDEN, 128), jnp.float32),
            pltpu.VMEM((HIDDEN, 128), jnp.float32),
            pltpu.SemaphoreType.DMA,
            pltpu.SemaphoreType.DMA,
            pltpu.SemaphoreType.DMA,
            pltpu.SemaphoreType.DMA,
            pltpu.SemaphoreType.DMA,
            pltpu.SemaphoreType.DMA,
        ],
    )(xt, wt2, pt)


def kernel(x, word_table, pos_table):
    xt = x.astype(jnp.int32).T
    pos200 = pos_table[:SEQ_LEN]
    posx = jnp.concatenate([pos200, pos200], axis=1)
    out_t = _run(xt, word_table, posx)
    return jnp.transpose(out_t, (2, 0, 1))


# speedup vs baseline: 1.0076x; 1.0076x over previous
import jax
import jax.numpy as jnp
from jax import lax
from jax.experimental import pallas as pl
from jax.experimental.pallas import tpu as pltpu
from jax.experimental.pallas import tpu_sc as plsc

BATCH = 4096
SEQ_LEN = 200
HIDDEN = 64
LANES = 16

NUM_CORES = 2
NUM_SUBCORES = 16
NUM_WORKERS = NUM_CORES * NUM_SUBCORES

LT = SEQ_LEN // 8

_TAKE_DNUMS = lax.GatherDimensionNumbers(
    offset_dims=(), collapsed_slice_dims=(0,), start_index_map=(0,)
)


def _take16(vec, idx):
    return lax.gather(
        vec,
        idx[:, None],
        _TAKE_DNUMS,
        slice_sizes=(1,),
        mode=lax.GatherScatterMode.PROMISE_IN_BOUNDS,
    )


def _body(xt_hbm, wt2_hbm, px_hbm, out_hbm, idx_v, idx2_v, posall_v,
          rows0, rows1, rows2, rows3, st0, st1,
          semg0, semg1, semg2, semg3, semo0, semo1):
    wid = lax.axis_index("s") * NUM_CORES + lax.axis_index("c")
    b0 = wid * 128

    rows = (rows0, rows1, rows2, rows3)
    stage = (st0, st1)
    semg = (semg0, semg1, semg2, semg3)
    semo = (semo0, semo1)

    iota = lax.iota(jnp.int32, LANES)
    rowvec = [iota + (16 * k) for k in range(8)]

    def issue_gather(j, b):
        pltpu.async_copy(wt2_hbm.at[idx2_v.at[j]], rows[b], semg[b])

    def wait_gather(b):
        pltpu.make_async_copy(wt2_hbm.at[pl.ds(0, 128)], rows[b], semg[b]).wait()

    def wait_out(b):
        pltpu.make_async_copy(stage[b], out_hbm.at[0, :, pl.ds(b0, 128)], semo[b]).wait()

    pltpu.sync_copy(px_hbm, posall_v)

    def lt_body(lt, carry):
        l0 = lt * 8
        pltpu.sync_copy(xt_hbm.at[pl.ds(l0, 8), pl.ds(b0, 128)], idx_v)
        for j in range(8):
            for k in range(8):
                v16 = idx_v[j, pl.ds(16 * k, 16)]
                idx2_v[j, pl.ds(16 * k, 16)] = lax.shift_right_logical(v16, 1)
        for j3 in range(3):
            issue_gather(j3, j3)
        for j in range(8):
            if j < 5:
                issue_gather(j + 3, (j + 3) % 4)
            b = j % 4
            s = j % 2
            wait_gather(b)
            if j >= 2:
                wait_out(s)
            else:
                @pl.when(lt > 0)
                def _():
                    wait_out(s)
            pv = [
                lax.shift_left(idx_v[j, pl.ds(16 * k, 16)] & 1, 6)
                for k in range(8)
            ]

            @plsc.parallel_loop(0, HIDDEN, unroll=4)
            def _diag_loop(i):
                d = i & 15
                h0 = lax.shift_right_logical(i, 4) * 16
                rot = (iota + d) & 15
                hvec = rot + h0
                povec = posall_v[l0 + j, pl.ds(h0, 16)]
                ps = _take16(povec, rot)
                for k in range(8):
                    col = pv[k] + hvec
                    vec = plsc.load_gather(rows[b], [rowvec[k], col])
                    plsc.store_scatter(stage[s], [hvec, rowvec[k]], vec + ps)
            pltpu.async_copy(stage[s], out_hbm.at[l0 + j, :, pl.ds(b0, 128)], semo[s])
        return carry

    lax.fori_loop(0, LT, lt_body, 0)
    wait_out(0)
    wait_out(1)


@jax.jit
def _run(xt, wt2, pt):
    mesh = plsc.VectorSubcoreMesh(core_axis_name="c", subcore_axis_name="s")
    return pl.kernel(
        _body,
        out_type=jax.ShapeDtypeStruct((SEQ_LEN, HIDDEN, BATCH), jnp.float32),
        mesh=mesh,
        compiler_params=pltpu.CompilerParams(
            use_tc_tiling_on_sc=True, needs_layout_passes=False
        ),
        scratch_types=[
            pltpu.VMEM((8, 128), jnp.int32),
            pltpu.VMEM((8, 128), jnp.int32),
            pltpu.VMEM((SEQ_LEN, 128), jnp.float32),
            pltpu.VMEM((128, 128), jnp.float32),
            pltpu.VMEM((128, 128), jnp.float32),
            pltpu.VMEM((128, 128), jnp.float32),
            pltpu.VMEM((128, 128), jnp.float32),
            pltpu.VMEM((HIDDEN, 128), jnp.float32),
            pltpu.VMEM((HIDDEN, 128), jnp.float32),
            pltpu.SemaphoreType.DMA,
            pltpu.SemaphoreType.DMA,
            pltpu.SemaphoreType.DMA,
            pltpu.SemaphoreType.DMA,
            pltpu.SemaphoreType.DMA,
            pltpu.SemaphoreType.DMA,
        ],
    )(xt, wt2, pt)


def kernel(x, word_table, pos_table):
    xt = x.astype(jnp.int32).T
    wt2 = word_table.reshape(500000, 128)
    pos200 = pos_table[:SEQ_LEN]
    posx = jnp.concatenate([pos200, pos200], axis=1)
    out_t = _run(xt, wt2, posx)
    return jnp.transpose(out_t, (2, 0, 1))


# speedup vs baseline: 1.9035x; 1.8891x over previous
import jax
import jax.numpy as jnp
from jax import lax
from jax.experimental import pallas as pl
from jax.experimental.pallas import tpu as pltpu
from jax.experimental.pallas import tpu_sc as plsc

BATCH = 4096
SEQ_LEN = 200
HIDDEN = 64
LANES = 16

NUM_CORES = 2
NUM_SUBCORES = 16
NUM_WORKERS = NUM_CORES * NUM_SUBCORES

LT = SEQ_LEN // 8

_TAKE_DNUMS = lax.GatherDimensionNumbers(
    offset_dims=(), collapsed_slice_dims=(0,), start_index_map=(0,)
)


def _take16(vec, idx):
    return lax.gather(
        vec,
        idx[:, None],
        _TAKE_DNUMS,
        slice_sizes=(1,),
        mode=lax.GatherScatterMode.PROMISE_IN_BOUNDS,
    )


NSLAB = 999936 // 256


def _transp_body(wtt_hbm, tail_hbm, t2_hbm, in0, in1, ot0, ot1,
                 semi0, semi1, semt0, semt1):
    wid = lax.axis_index("s") * NUM_CORES + lax.axis_index("c")
    inb = (in0, in1)
    otb = (ot0, ot1)
    semi = (semi0, semi1)
    semt = (semt0, semt1)
    iota = lax.iota(jnp.int32, LANES)
    rowjb = [16 * (jb & 3) + iota for jb in range(8)]

    def slab_of(i):
        return wid + i * NUM_WORKERS

    def issue_in(i, b):
        s = slab_of(i)

        @pl.when(s < NSLAB)
        def _():
            pltpu.async_copy(wtt_hbm.at[:, pl.ds(s * 256, 256)], inb[b], semi[b])

    def wait_in(i, b):
        s = slab_of(i)

        @pl.when(s < NSLAB)
        def _():
            pltpu.make_async_copy(wtt_hbm.at[:, pl.ds(0, 256)], inb[b], semi[b]).wait()

    def wait_out(i, b):
        s = slab_of(i)

        @pl.when(s < NSLAB)
        def _():
            pltpu.make_async_copy(otb[b], t2_hbm.at[pl.ds(0, 128)], semt[b]).wait()

    issue_in(0, 0)
    issue_in(1, 1)

    def super_body(sp, carry):
        for b in range(2):
            i = 2 * sp + b
            s = slab_of(i)
            wait_in(i, b)

            @pl.when(i >= 2)
            def _():
                wait_out(i - 2, b)

            @pl.when(s < NSLAB)
            def _():
                @plsc.parallel_loop(0, 128, unroll=2)
                def _blk(t):
                    d = t & 15
                    pb16 = lax.shift_right_logical(t, 4) * 16
                    rot = (iota + d) & 15
                    p_idx = rot + pb16
                    cbase = 2 * pb16 + 2 * rot
                    for jb in range(8):
                        col = cbase + (jb >> 2)
                        vec = plsc.load_gather(inb[b], [rowjb[jb], col])
                        plsc.store_scatter(otb[b], [p_idx, 16 * jb + iota], vec)

                pltpu.async_copy(otb[b], t2_hbm.at[pl.ds(s * 128, 128)], semt[b])

            issue_in(i + 2, b)
        return carry

    lax.fori_loop(0, 62, super_body, 0)
    wait_out(122, 0)

    @pl.when(wid == 0)
    def _():
        pltpu.sync_copy(tail_hbm, ot0.at[pl.ds(0, 32)])
        pltpu.sync_copy(ot0.at[pl.ds(0, 32)], t2_hbm.at[pl.ds(499968, 32)])


def _body(xt_hbm, wt2_hbm, px_hbm, out_hbm, idx_v, idx2_v, posall_v,
          rows0, rows1, rows2, rows3, st0, st1,
          semg0, semg1, semg2, semg3, semo0, semo1):
    wid = lax.axis_index("s") * NUM_CORES + lax.axis_index("c")
    b0 = wid * 128

    rows = (rows0, rows1, rows2, rows3)
    stage = (st0, st1)
    semg = (semg0, semg1, semg2, semg3)
    semo = (semo0, semo1)

    iota = lax.iota(jnp.int32, LANES)
    rowvec = [iota + (16 * k) for k in range(8)]

    def issue_gather(j, b):
        pltpu.async_copy(wt2_hbm.at[idx2_v.at[j]], rows[b], semg[b])

    def wait_gather(b):
        pltpu.make_async_copy(wt2_hbm.at[pl.ds(0, 128)], rows[b], semg[b]).wait()

    def wait_out(b):
        pltpu.make_async_copy(stage[b], out_hbm.at[0, :, pl.ds(b0, 128)], semo[b]).wait()

    pltpu.sync_copy(px_hbm, posall_v)

    def lt_body(lt, carry):
        l0 = lt * 8
        pltpu.sync_copy(xt_hbm.at[pl.ds(l0, 8), pl.ds(b0, 128)], idx_v)
        for j in range(8):
            for k in range(8):
                v16 = idx_v[j, pl.ds(16 * k, 16)]
                idx2_v[j, pl.ds(16 * k, 16)] = lax.shift_right_logical(v16, 1)
        for j3 in range(3):
            issue_gather(j3, j3)
        for j in range(8):
            if j < 5:
                issue_gather(j + 3, (j + 3) % 4)
            b = j % 4
            s = j % 2
            wait_gather(b)
            if j >= 2:
                wait_out(s)
            else:
                @pl.when(lt > 0)
                def _():
                    wait_out(s)
            pv = [
                lax.shift_left(idx_v[j, pl.ds(16 * k, 16)] & 1, 6)
                for k in range(8)
            ]

            @plsc.parallel_loop(0, HIDDEN, unroll=4)
            def _diag_loop(i):
                d = i & 15
                h0 = lax.shift_right_logical(i, 4) * 16
                rot = (iota + d) & 15
                hvec = rot + h0
                povec = posall_v[l0 + j, pl.ds(h0, 16)]
                ps = _take16(povec, rot)
                for k in range(8):
                    col = pv[k] + hvec
                    vec = plsc.load_gather(rows[b], [rowvec[k], col])
                    plsc.store_scatter(stage[s], [hvec, rowvec[k]], vec + ps)
            pltpu.async_copy(stage[s], out_hbm.at[l0 + j, :, pl.ds(b0, 128)], semo[s])
        return carry

    lax.fori_loop(0, LT, lt_body, 0)
    wait_out(0)
    wait_out(1)


@jax.jit
def _run(xt, wtt, tail2, pt):
    mesh = plsc.VectorSubcoreMesh(core_axis_name="c", subcore_axis_name="s")
    wt2 = pl.kernel(
        _transp_body,
        out_type=jax.ShapeDtypeStruct((500000, 128), jnp.float32),
        mesh=mesh,
        compiler_params=pltpu.CompilerParams(
            use_tc_tiling_on_sc=True, needs_layout_passes=False
        ),
        scratch_types=[
            pltpu.VMEM((HIDDEN, 256), jnp.float32),
            pltpu.VMEM((HIDDEN, 256), jnp.float32),
            pltpu.VMEM((128, 128), jnp.float32),
            pltpu.VMEM((128, 128), jnp.float32),
            pltpu.SemaphoreType.DMA,
            pltpu.SemaphoreType.DMA,
            pltpu.SemaphoreType.DMA,
            pltpu.SemaphoreType.DMA,
        ],
    )(wtt, tail2)
    return pl.kernel(
        _body,
        out_type=jax.ShapeDtypeStruct((SEQ_LEN, HIDDEN, BATCH), jnp.float32),
        mesh=mesh,
        compiler_params=pltpu.CompilerParams(
            use_tc_tiling_on_sc=True, needs_layout_passes=False
        ),
        scratch_types=[
            pltpu.VMEM((8, 128), jnp.int32),
            pltpu.VMEM((8, 128), jnp.int32),
            pltpu.VMEM((SEQ_LEN, 128), jnp.float32),
            pltpu.VMEM((128, 128), jnp.float32),
            pltpu.VMEM((128, 128), jnp.float32),
            pltpu.VMEM((128, 128), jnp.float32),
            pltpu.VMEM((128, 128), jnp.float32),
            pltpu.VMEM((HIDDEN, 128), jnp.float32),
            pltpu.VMEM((HIDDEN, 128), jnp.float32),
            pltpu.SemaphoreType.DMA,
            pltpu.SemaphoreType.DMA,
            pltpu.SemaphoreType.DMA,
            pltpu.SemaphoreType.DMA,
            pltpu.SemaphoreType.DMA,
            pltpu.SemaphoreType.DMA,
        ],
    )(xt, wt2, pt)


def kernel(x, word_table, pos_table):
    xt = x.astype(jnp.int32).T
    wtt = word_table.T
    tail2 = word_table[999936:].reshape(32, 128)
    pos200 = pos_table[:SEQ_LEN]
    posx = jnp.concatenate([pos200, pos200], axis=1)
    out_t = _run(xt, wtt, tail2, posx)
    return jnp.transpose(out_t, (2, 0, 1))
